# Initial kernel scaffold; baseline (speedup 1.0000x reference)
import jax
import jax.numpy as jnp
from jax.experimental import pallas as pl


def kernel(x, pos_table, gamma, beta):
    raise NotImplementedError("write your pallas kernel here")


# speedup vs baseline: 2.9916x; 2.9916x over previous
import jax
import jax.numpy as jnp
from jax.experimental import pallas as pl

B = 64
N_PATCHES = 1024
HIDDEN = 768

BN = 512


def _ln_kernel(x_ref, pos_ref, gamma_ref, beta_ref, out_ref):
    h = x_ref[...] + pos_ref[...]
    mean = jnp.mean(h, axis=1, keepdims=True)
    c = h - mean
    var = jnp.mean(c * c, axis=1, keepdims=True)
    inv = jax.lax.rsqrt(var + 1e-12)
    out_ref[...] = (c * inv) * gamma_ref[...] + beta_ref[...]


def kernel(x, pos_table, gamma, beta):
    Bx, n_patches, hidden = x.shape
    x2 = x.reshape(Bx * n_patches, hidden)
    gamma2 = gamma.reshape(1, hidden)
    beta2 = beta.reshape(1, hidden)
    n_pos_blocks = n_patches // BN
    grid = (Bx * n_patches // BN,)
    out = pl.pallas_call(
        _ln_kernel,
        grid=grid,
        in_specs=[
            pl.BlockSpec((BN, hidden), lambda i: (i, 0)),
            pl.BlockSpec((BN, hidden), lambda i: (i % n_pos_blocks, 0)),
            pl.BlockSpec((1, hidden), lambda i: (0, 0)),
            pl.BlockSpec((1, hidden), lambda i: (0, 0)),
        ],
        out_specs=pl.BlockSpec((BN, hidden), lambda i: (i, 0)),
        out_shape=jax.ShapeDtypeStruct((Bx * n_patches, hidden), x.dtype),
    )(x2, pos_table, gamma2, beta2)
    return out.reshape(Bx, n_patches, hidden)


# speedup vs baseline: 2.9964x; 1.0016x over previous
import jax
import jax.numpy as jnp
from jax.experimental import pallas as pl
from jax.experimental.pallas import tpu as pltpu

B = 64
N_PATCHES = 1024
HIDDEN = 768

BN = 512


def _ln_kernel(x_ref, pos_ref, gamma_ref, beta_ref, out_ref):
    h = x_ref[...] + pos_ref[...]
    mean = jnp.mean(h, axis=1, keepdims=True)
    c = h - mean
    var = jnp.mean(c * c, axis=1, keepdims=True)
    inv = jax.lax.rsqrt(var + 1e-12)
    out_ref[...] = (c * inv) * gamma_ref[...] + beta_ref[...]


def kernel(x, pos_table, gamma, beta):
    Bx, n_patches, hidden = x.shape
    x2 = x.reshape(Bx * n_patches, hidden)
    gamma2 = gamma.reshape(1, hidden)
    beta2 = beta.reshape(1, hidden)
    n_pos_blocks = n_patches // BN
    grid = (Bx * n_patches // BN,)
    out = pl.pallas_call(
        _ln_kernel,
        grid=grid,
        in_specs=[
            pl.BlockSpec((BN, hidden), lambda i: (i, 0)),
            pl.BlockSpec((BN, hidden), lambda i: (i % n_pos_blocks, 0)),
            pl.BlockSpec((1, hidden), lambda i: (0, 0)),
            pl.BlockSpec((1, hidden), lambda i: (0, 0)),
        ],
        out_specs=pl.BlockSpec((BN, hidden), lambda i: (i, 0)),
        out_shape=jax.ShapeDtypeStruct((Bx * n_patches, hidden), x.dtype),
        compiler_params=pltpu.CompilerParams(
            dimension_semantics=("parallel",),
        ),
    )(x2, pos_table, gamma2, beta2)
    return out.reshape(Bx, n_patches, hidden)


# speedup vs baseline: 4.4985x; 1.5013x over previous
import jax
import jax.numpy as jnp
from jax.experimental import pallas as pl
from jax.experimental.pallas import tpu as pltpu

B = 64
N_PATCHES = 1024
HIDDEN = 768

BN = 1024


def _ln_kernel(x_ref, pos_ref, gamma_ref, beta_ref, out_ref):
    h = x_ref[...] + pos_ref[...]
    mean = jnp.mean(h, axis=1, keepdims=True)
    c = h - mean
    var = jnp.mean(c * c, axis=1, keepdims=True)
    inv = jax.lax.rsqrt(var + 1e-12)
    out_ref[...] = (c * inv) * gamma_ref[...] + beta_ref[...]


def kernel(x, pos_table, gamma, beta):
    Bx, n_patches, hidden = x.shape
    x2 = x.reshape(Bx * n_patches, hidden)
    gamma2 = gamma.reshape(1, hidden)
    beta2 = beta.reshape(1, hidden)
    n_pos_blocks = n_patches // BN
    grid = (Bx * n_patches // BN,)
    out = pl.pallas_call(
        _ln_kernel,
        grid=grid,
        in_specs=[
            pl.BlockSpec((BN, hidden), lambda i: (i, 0)),
            pl.BlockSpec((BN, hidden), lambda i: (i % n_pos_blocks, 0)),
            pl.BlockSpec((1, hidden), lambda i: (0, 0)),
            pl.BlockSpec((1, hidden), lambda i: (0, 0)),
        ],
        out_specs=pl.BlockSpec((BN, hidden), lambda i: (i, 0)),
        out_shape=jax.ShapeDtypeStruct((Bx * n_patches, hidden), x.dtype),
        compiler_params=pltpu.CompilerParams(
            dimension_semantics=("parallel",),
        ),
    )(x2, pos_table, gamma2, beta2)
    return out.reshape(Bx, n_patches, hidden)


# speedup vs baseline: 4.8653x; 1.0815x over previous
import jax
import jax.numpy as jnp
from jax.experimental import pallas as pl
from jax.experimental.pallas import tpu as pltpu

B = 64
N_PATCHES = 1024
HIDDEN = 768

BN = 2048


def _ln_kernel(x_ref, pos_ref, gamma_ref, beta_ref, out_ref):
    xv = x_ref[...]
    reps = xv.shape[0] // pos_ref.shape[0]
    pv = pos_ref[...]
    h = (xv.reshape(reps, pos_ref.shape[0], xv.shape[1]) + pv[None]).reshape(xv.shape)
    mean = jnp.mean(h, axis=1, keepdims=True)
    c = h - mean
    var = jnp.mean(c * c, axis=1, keepdims=True)
    inv = jax.lax.rsqrt(var + 1e-12)
    out_ref[...] = (c * inv) * gamma_ref[...] + beta_ref[...]


def kernel(x, pos_table, gamma, beta):
    Bx, n_patches, hidden = x.shape
    x2 = x.reshape(Bx * n_patches, hidden)
    gamma2 = gamma.reshape(1, hidden)
    beta2 = beta.reshape(1, hidden)
    grid = (Bx * n_patches // BN,)
    out = pl.pallas_call(
        _ln_kernel,
        grid=grid,
        in_specs=[
            pl.BlockSpec((BN, hidden), lambda i: (i, 0)),
            pl.BlockSpec((n_patches, hidden), lambda i: (0, 0)),
            pl.BlockSpec((1, hidden), lambda i: (0, 0)),
            pl.BlockSpec((1, hidden), lambda i: (0, 0)),
        ],
        out_specs=pl.BlockSpec((BN, hidden), lambda i: (i, 0)),
        out_shape=jax.ShapeDtypeStruct((Bx * n_patches, hidden), x.dtype),
        compiler_params=pltpu.CompilerParams(
            dimension_semantics=("parallel",),
        ),
    )(x2, pos_table, gamma2, beta2)
    return out.reshape(Bx, n_patches, hidden)


# speedup vs baseline: 4.9354x; 1.0144x over previous
import jax
import jax.numpy as jnp
from jax.experimental import pallas as pl
from jax.experimental.pallas import tpu as pltpu

BN = 3072


def _ln_kernel(x_ref, pos_ref, gamma_ref, beta_ref, out_ref):
    xv = x_ref[...]
    np_rows = pos_ref.shape[0]
    reps = xv.shape[0] // np_rows
    pv = pos_ref[...]
    h = (xv.reshape(reps, np_rows, xv.shape[1]) + pv[None]).reshape(xv.shape)
    mean = jnp.mean(h, axis=1, keepdims=True)
    c = h - mean
    var = jnp.mean(c * c, axis=1, keepdims=True)
    inv = jax.lax.rsqrt(var + 1e-12)
    out_ref[...] = (c * inv) * gamma_ref[...] + beta_ref[...]


def kernel(x, pos_table, gamma, beta):
    Bx, n_patches, hidden = x.shape
    rows = Bx * n_patches
    x2 = x.reshape(rows, hidden)
    gamma2 = gamma.reshape(1, hidden)
    beta2 = beta.reshape(1, hidden)
    grid = (pl.cdiv(rows, BN),)
    out = pl.pallas_call(
        _ln_kernel,
        grid=grid,
        in_specs=[
            pl.BlockSpec((BN, hidden), lambda i: (i, 0)),
            pl.BlockSpec((n_patches, hidden), lambda i: (0, 0)),
            pl.BlockSpec((1, hidden), lambda i: (0, 0)),
            pl.BlockSpec((1, hidden), lambda i: (0, 0)),
        ],
        out_specs=pl.BlockSpec((BN, hidden), lambda i: (i, 0)),
        out_shape=jax.ShapeDtypeStruct((rows, hidden), x.dtype),
        compiler_params=pltpu.CompilerParams(
            dimension_semantics=("parallel",),
        ),
    )(x2, pos_table, gamma2, beta2)
    return out.reshape(Bx, n_patches, hidden)
